# baseline (speedup 1.0000x reference)
import jax
import jax.numpy as jnp
from jax import lax
from jax.experimental import pallas as pl
from jax.experimental.pallas import tpu as pltpu
from jax.experimental.pallas import tpu_sc as plsc

N_CLS = 1000
N_CTX = 16
D = 768
CTX_LEN = 77
SUF = CTX_LEN - 1 - N_CTX
NW = 32
SUF_ROWS = N_CLS * SUF
SUF_CHUNK = 1880
SUF_LAST = SUF_ROWS - (NW - 1) * SUF_CHUNK
PRE_CHUNK = 32
PRE_LAST = N_CLS - (NW - 1) * PRE_CHUNK
REP = 128
HALF0 = 504
HALF1 = 496


def _body(prefix_hbm, suffix_t_hbm, ctx_hbm, idx_tab_hbm, out_hbm, rep_v, idx_v, sem):
    wid = lax.axis_index("s") * 2 + lax.axis_index("c")

    @pl.when(wid < NW - 1)
    def _suf_main():
        off = pl.multiple_of(wid * SUF_CHUNK, 8)
        pltpu.async_copy(
            suffix_t_hbm.at[pl.ds(off, SUF_CHUNK)],
            out_hbm.at[pl.ds(pl.multiple_of(N_CLS * (1 + N_CTX) + off, 8), SUF_CHUNK)],
            sem,
        )

    @pl.when(wid == NW - 1)
    def _suf_tail():
        off = (NW - 1) * SUF_CHUNK
        pltpu.async_copy(
            suffix_t_hbm.at[pl.ds(off, SUF_LAST)],
            out_hbm.at[pl.ds(N_CLS * (1 + N_CTX) + off, SUF_LAST)],
            sem,
        )

    @pl.when(wid < NW - 1)
    def _pre_main():
        off = pl.multiple_of(wid * PRE_CHUNK, 8)
        pltpu.async_copy(
            prefix_hbm.at[pl.ds(off, PRE_CHUNK)], out_hbm.at[pl.ds(off, PRE_CHUNK)], sem
        )

    @pl.when(wid == NW - 1)
    def _pre_tail():
        off = (NW - 1) * PRE_CHUNK
        pltpu.async_copy(
            prefix_hbm.at[pl.ds(off, PRE_LAST)], out_hbm.at[pl.ds(off, PRE_LAST)], sem
        )

    pltpu.sync_copy(idx_tab_hbm.at[wid], idx_v)
    pltpu.sync_copy(ctx_hbm.at[idx_v], rep_v)
    t = lax.div(wid, 2)
    h = lax.rem(wid, 2)
    slab = N_CLS * (1 + t)

    @pl.when(h == 0)
    def _ctx0():
        base = pl.multiple_of(slab, 8)
        for k in range(3):
            pltpu.async_copy(rep_v, out_hbm.at[pl.ds(base + k * REP, REP)], sem)
        pltpu.async_copy(
            rep_v.at[pl.ds(0, HALF0 - 3 * REP)],
            out_hbm.at[pl.ds(base + 3 * REP, HALF0 - 3 * REP)],
            sem,
        )

    @pl.when(h == 1)
    def _ctx1():
        base = pl.multiple_of(slab + HALF0, 8)
        for k in range(3):
            pltpu.async_copy(rep_v, out_hbm.at[pl.ds(base + k * REP, REP)], sem)
        pltpu.async_copy(
            rep_v.at[pl.ds(0, HALF1 - 3 * REP)],
            out_hbm.at[pl.ds(base + 3 * REP, HALF1 - 3 * REP)],
            sem,
        )

    @pl.when(wid < NW - 1)
    def _drain_main():
        pltpu.make_async_copy(
            suffix_t_hbm.at[pl.ds(0, SUF_CHUNK)],
            out_hbm.at[pl.ds(0, SUF_CHUNK)],
            sem,
        ).wait()
        pltpu.make_async_copy(
            prefix_hbm.at[pl.ds(0, PRE_CHUNK)], out_hbm.at[pl.ds(0, PRE_CHUNK)], sem
        ).wait()

    @pl.when(wid == NW - 1)
    def _drain_tail():
        pltpu.make_async_copy(
            suffix_t_hbm.at[pl.ds(0, SUF_LAST)], out_hbm.at[pl.ds(0, SUF_LAST)], sem
        ).wait()
        pltpu.make_async_copy(
            prefix_hbm.at[pl.ds(0, PRE_LAST)], out_hbm.at[pl.ds(0, PRE_LAST)], sem
        ).wait()

    @pl.when(h == 0)
    def _drain_ctx0():
        for _ in range(3):
            pltpu.make_async_copy(rep_v, out_hbm.at[pl.ds(0, REP)], sem).wait()
        pltpu.make_async_copy(
            rep_v.at[pl.ds(0, HALF0 - 3 * REP)],
            out_hbm.at[pl.ds(0, HALF0 - 3 * REP)],
            sem,
        ).wait()

    @pl.when(h == 1)
    def _drain_ctx1():
        for _ in range(3):
            pltpu.make_async_copy(rep_v, out_hbm.at[pl.ds(0, REP)], sem).wait()
        pltpu.make_async_copy(
            rep_v.at[pl.ds(0, HALF1 - 3 * REP)],
            out_hbm.at[pl.ds(0, HALF1 - 3 * REP)],
            sem,
        ).wait()


def kernel(ctx, prefix_embedding, suffix_embedding):
    prefix_flat = prefix_embedding.reshape(N_CLS, D)
    suffix_t = suffix_embedding.transpose(1, 0, 2).reshape(SUF_ROWS, D)
    idx_tab = jnp.broadcast_to(
        (jnp.arange(NW, dtype=jnp.int32) // 2)[:, None], (NW, REP)
    )
    mesh = plsc.VectorSubcoreMesh(core_axis_name="c", subcore_axis_name="s")
    k = pl.kernel(
        _body,
        out_type=jax.ShapeDtypeStruct((N_CLS * CTX_LEN, D), jnp.float32),
        mesh=mesh,
        scratch_types=[
            pltpu.VMEM((REP, D), jnp.float32),
            pltpu.VMEM((REP,), jnp.int32),
            pltpu.SemaphoreType.DMA,
        ],
    )
    out = k(prefix_flat, suffix_t, ctx, idx_tab)
    return out.reshape(CTX_LEN, N_CLS, D).transpose(1, 0, 2)


# speedup vs baseline: 8.0175x; 8.0175x over previous
import jax
import jax.numpy as jnp
from jax import lax
from jax.experimental import pallas as pl
from jax.experimental.pallas import tpu as pltpu
from jax.experimental.pallas import tpu_sc as plsc

N_CLS = 1000
N_CTX = 16
D = 768
CTX_LEN = 77
SUF = CTX_LEN - 1 - N_CTX
SUF_MAIN = 56
TAIL = 8
NW = 32
PER_W = 32


def _body(
    ctx_hbm,
    prefix_hbm,
    suffix_hbm,
    suffix_flat_hbm,
    idx_ctx_hbm,
    idx_suf_hbm,
    idx_tail_even_hbm,
    idx_tail_odd_hbm,
    out_hbm,
    ctx_v,
    suf_v,
    tail_v,
    idx_ctx_v,
    idx_suf_v,
    idx_tail_even_v,
    idx_tail_odd_v,
):
    wid = lax.axis_index("s") * 2 + lax.axis_index("c")
    pltpu.sync_copy(ctx_hbm, ctx_v)
    pltpu.sync_copy(idx_ctx_hbm, idx_ctx_v)
    pltpu.sync_copy(idx_suf_hbm, idx_suf_v)
    pltpu.sync_copy(idx_tail_even_hbm, idx_tail_even_v)
    pltpu.sync_copy(idx_tail_odd_hbm, idx_tail_odd_v)

    def cls_body(i, carry):
        c = wid * PER_W + i

        @pl.when(c < N_CLS)
        def _do():
            out_c = out_hbm.at[c]
            pltpu.sync_copy(suffix_hbm.at[c].at[pl.ds(0, SUF_MAIN)], suf_v)
            pltpu.sync_copy(suf_v, out_c.at[idx_suf_v])

            is_odd = lax.rem(c, 2)
            off = pl.multiple_of(c * SUF + SUF_MAIN - 4 * is_odd, 8)
            pltpu.sync_copy(suffix_flat_hbm.at[pl.ds(off, TAIL)], tail_v)

            @pl.when(is_odd == 0)
            def _even():
                pltpu.sync_copy(tail_v, out_c.at[idx_tail_even_v])

            @pl.when(is_odd == 1)
            def _odd():
                pltpu.sync_copy(tail_v, out_c.at[idx_tail_odd_v])

            pltpu.sync_copy(ctx_v, out_c.at[idx_ctx_v])
            pltpu.sync_copy(prefix_hbm.at[c], out_c.at[pl.ds(0, 1)])

        return carry

    lax.fori_loop(0, PER_W, cls_body, 0)


def kernel(ctx, prefix_embedding, suffix_embedding):
    suffix_flat = suffix_embedding.reshape(N_CLS * SUF, D)
    idx_ctx = jnp.arange(1, 1 + N_CTX, dtype=jnp.int32)
    idx_suf = jnp.arange(1 + N_CTX, 1 + N_CTX + SUF_MAIN, dtype=jnp.int32)
    idx_tail_even = jnp.array([73, 74, 75, 76, 1, 2, 3, 4], dtype=jnp.int32)
    idx_tail_odd = jnp.arange(CTX_LEN - TAIL, CTX_LEN, dtype=jnp.int32)
    mesh = plsc.VectorSubcoreMesh(core_axis_name="c", subcore_axis_name="s")
    k = pl.kernel(
        _body,
        out_type=jax.ShapeDtypeStruct((N_CLS, CTX_LEN, D), jnp.float32),
        mesh=mesh,
        scratch_types=[
            pltpu.VMEM((N_CTX, D), jnp.float32),
            pltpu.VMEM((SUF_MAIN, D), jnp.float32),
            pltpu.VMEM((TAIL, D), jnp.float32),
            pltpu.VMEM((N_CTX,), jnp.int32),
            pltpu.VMEM((SUF_MAIN,), jnp.int32),
            pltpu.VMEM((TAIL,), jnp.int32),
            pltpu.VMEM((TAIL,), jnp.int32),
        ],
    )
    return k(
        ctx,
        prefix_embedding,
        suffix_embedding,
        suffix_flat,
        idx_ctx,
        idx_suf,
        idx_tail_even,
        idx_tail_odd,
    )
